# baseline (speedup 1.0000x reference)
import functools

import jax
import jax.numpy as jnp
import numpy as np
from jax import lax
from jax.experimental import pallas as pl
from jax.experimental.pallas import tpu as pltpu
from jax.experimental.pallas import tpu_sc as plsc

N_NODES = 10000
N_EDGES = 160000
B = 64
NF = 128
EDGE_BINS = 128
TRIP = 40
PDOS = 200

_NC = 2
_NS = 16
_NW = _NC * _NS
_CH = 40
_NPAD = 10240
_ROWS_PW = _NPAD // _NS

_EBLOCKS = (30720, 98560, 30720)

_F32 = jnp.float32
_BF16 = jnp.bfloat16


def _softplus(x):
    return jnp.log1p(jnp.exp(-jnp.abs(x))) + jnp.maximum(x, 0.0)


def _silu(x):
    return x * jax.nn.sigmoid(x)


def _lnorm(x, g, b, eps=1e-5):
    m = jnp.mean(x, axis=-1, keepdims=True)
    v = jnp.mean((x - m) ** 2, axis=-1, keepdims=True)
    return (x - m) * lax.rsqrt(v + eps) * g + b


def _rbf_rows(col, vmin, vmax, bins):
    step = (vmax - vmin) / (bins - 1)
    centers = vmin + step * lax.broadcasted_iota(jnp.int32, (1, bins), 1).astype(_F32)
    gamma = 1.0 / (step * step)
    return jnp.exp(-gamma * (col - centers) ** 2)


def _lat_body(len_ref, ang_ref, rw1, rb1, rw2, rb2, aw1, ab1, aw2, ab2,
              elen_ref, eang_ref):
    r = _rbf_rows(len_ref[...], 0.0, 8.0, EDGE_BINS)
    elen_ref[...] = _softplus(r @ rw1[...] + rb1[...][None]) @ rw2[...] + rb2[...][None]
    a = _rbf_rows(ang_ref[...], -1.0, 1.0, TRIP)
    eang_ref[...] = _softplus(a @ aw1[...] + ab1[...][None]) @ aw2[...] + ab2[...][None]


def _latemb_body(cat_ref, w1, b1, w2, b2, out_ref):
    h = _softplus(cat_ref[...] @ w1[...] + b1[...][None]) @ w2[...] + b2[...][None]
    out_ref[...] = h


def _prelude_body(x_ref, b_ref, lat_ref, aw, ab, w1, b1, w2, b2, out_ref):
    nf0 = x_ref[...] @ aw[...] + ab[...][None]
    rows = x_ref.shape[0]
    onehot = (b_ref[...] == lax.broadcasted_iota(jnp.int32, (rows, B), 1).astype(_F32)).astype(_F32)
    le = onehot @ lat_ref[...]
    h = jnp.concatenate([nf0, le], axis=-1)
    out_ref[...] = _softplus(h @ w1[...] + b1[...][None]) @ w2[...] + b2[...][None]


def _edgefeat_body(d2_ref, rw1, rb1c, rw2, rb2c, we1, be1, we2, be2,
                   e1_ref, e2_ref):
    d = jnp.sqrt(d2_ref[...])
    step = 8.0 / (EDGE_BINS - 1)
    centers = step * lax.broadcasted_iota(jnp.int32, (EDGE_BINS, 1), 0).astype(_F32)
    gamma = 1.0 / (step * step)
    rt = jnp.exp(-gamma * (d - centers) ** 2)
    ht = _softplus(jax.lax.dot_general(
        rw1[...], rt, (((0,), (0,)), ((), ())),
        preferred_element_type=_F32) + rb1c[...])
    eft = jax.lax.dot_general(
        rw2[...], ht, (((0,), (0,)), ((), ())),
        preferred_element_type=_F32) + rb2c[...]
    e1 = jax.lax.dot_general(eft, we1[...], (((0,), (0,)), ((), ())),
                             preferred_element_type=_F32)
    e2 = jax.lax.dot_general(eft, we2[...], (((0,), (0,)), ((), ())),
                             preferred_element_type=_F32)
    e1_ref[...] = (e1 + be1[...][None]).astype(jnp.bfloat16)
    e2_ref[...] = (e2 + be2[...][None]).astype(jnp.bfloat16)


def _qkv_body(nf_ref, wq, bq, wk, bk, wv, bv, td_ref, ts_ref):
    nf = nf_ref[...]
    q = nf @ wq[...] + bq[...][None]
    k = nf @ wk[...] + bk[...][None]
    v = nf @ wv[...] + bv[...][None]
    td_ref[...] = jnp.concatenate([q, q * k, v], axis=-1)
    ts_ref[...] = jnp.concatenate([k, v], axis=-1)


def _edgecompute_body(gd_ref, gs_ref, e_ref, wmu, bmu, wm, bm,
                      lag, lab, lmg, lmb, out_ref):
    gd = gd_ref[...]
    gs = gs_ref[...]
    e_b = e_ref[...]
    q_i = gd[:, :NF]
    qk_i = gd[:, NF:2 * NF]
    v_i = gd[:, 2 * NF:]
    k_j = gs[:, :NF]
    v_j = gs[:, NF:]
    e = e_b.astype(_F32)
    alpha = jnp.concatenate([qk_i, q_i * k_j, q_i * e], axis=-1)
    alpha = alpha * np.float32(1.0 / np.sqrt(3.0 * NF))
    gate = jax.nn.sigmoid(_lnorm(alpha, lag[...][None], lab[...][None]))
    msg = jnp.dot(jnp.concatenate([v_i.astype(_BF16), v_j.astype(_BF16),
                                   e_b], axis=-1), wmu[...],
                  preferred_element_type=_F32) + bmu[...][None]
    msg = msg * gate
    h = jnp.dot(msg.astype(jnp.bfloat16), wm[...],
                preferred_element_type=_F32) + bm[...][None]
    out_ref[...] = _lnorm(h, lmg[...][None], lmb[...][None])


def _make_post_body(n_blocks):
    def body(*args):
        agg_refs = args[:n_blocks]
        wc, bc, bng, bnb, out_ref = args[n_blocks:]
        agg = agg_refs[0][0] + agg_refs[0][1]
        for a in agg_refs[1:]:
            agg = agg + a[0] + a[1]
        o = agg @ wc[...] + bc[...][None]
        m = jnp.mean(o, axis=0, keepdims=True)
        v = jnp.mean((o - m) ** 2, axis=0, keepdims=True)
        o = (o - m) * lax.rsqrt(v + 1e-5) * bng[...][None] + bnb[...][None]
        out_ref[...] = _silu(o)
    return body


def _readout_body(nf_ref, brow_ref, lat_ref, fcw, fcb, pw, pb, out_ref):
    nf = nf_ref[...]
    onehot_t = (brow_ref[...] ==
                lax.broadcasted_iota(jnp.int32, (B, N_NODES), 0).astype(_F32)
                ).astype(_F32)
    seg = onehot_t @ nf
    cnt = jnp.sum(onehot_t, axis=1, keepdims=True)
    feats = seg / jnp.maximum(cnt, 1.0) + lat_ref[...]
    h = _silu(feats @ fcw[...] + fcb[...][None])
    out_ref[...] = h @ pw[...] + pb[...][None]


def _tc_call(body, out_shapes, grid=None, in_specs=None, out_specs=None):
    kwargs = {}
    if grid is not None:
        kwargs["grid"] = grid
    if in_specs is not None:
        kwargs["in_specs"] = in_specs
    if out_specs is not None:
        kwargs["out_specs"] = out_specs
    return pl.pallas_call(body, out_shape=out_shapes, **kwargs)


@functools.cache
def _sc_mesh():
    return plsc.VectorSubcoreMesh(core_axis_name="c", subcore_axis_name="s",
                                  num_cores=_NC, num_subcores=_NS)


def _pipelined(issue, drain, nchunk):
    issue(0, 0)
    npair = (nchunk - 1) // 2 if nchunk % 2 else (nchunk - 2) // 2

    @pl.loop(0, npair)
    def pair(j):
        i0 = 2 * j
        issue(i0 + 1, 1)
        drain(i0, 0)
        issue(i0 + 2, 0)
        drain(i0 + 1, 1)

    if nchunk % 2:
        drain(nchunk - 1, 0)
    else:
        issue(nchunk - 1, 1)
        drain(nchunk - 2, 0)
        drain(nchunk - 1, 1)


@functools.cache
def _sc_gather_kernel(n_edges):
    epw = n_edges // _NW
    nchunk = epw // _CH

    @functools.partial(
        pl.kernel,
        out_type=(
            jax.ShapeDtypeStruct((n_edges, 3 * NF), _F32),
            jax.ShapeDtypeStruct((n_edges, 2 * NF), _F32),
        ),
        mesh=_sc_mesh(),
        scratch_types=[
            pltpu.VMEM((nchunk, _CH), jnp.int32),
            pltpu.VMEM((nchunk, _CH), jnp.int32),
            pltpu.VMEM((2, _CH, 3 * NF), _F32),
            pltpu.VMEM((2, _CH, 2 * NF), _F32),
            pltpu.SemaphoreType.DMA,
            pltpu.SemaphoreType.DMA,
            pltpu.SemaphoreType.DMA,
            pltpu.SemaphoreType.DMA,
        ],
    )
    def gather(td_hbm, ts_hbm, dst_hbm, src_hbm, gd_hbm, gs_hbm,
               idx_d, idx_s, rows_d, rows_s, sd0, sd1, ss0, ss1):
        wid = lax.axis_index("s") * _NC + lax.axis_index("c")
        base = wid * epw
        pltpu.sync_copy(dst_hbm.at[wid], idx_d)
        pltpu.sync_copy(src_hbm.at[wid], idx_s)
        sem_d = (sd0, sd1)
        sem_s = (ss0, ss1)

        def issue(i, b):
            pltpu.async_copy(td_hbm.at[idx_d.at[i]], rows_d.at[b], sem_d[b])
            pltpu.async_copy(ts_hbm.at[idx_s.at[i]], rows_s.at[b], sem_s[b])

        def drain_and_store(i, b):
            pltpu.make_async_copy(td_hbm.at[idx_d.at[i]], rows_d.at[b],
                                  sem_d[b]).wait()
            pltpu.make_async_copy(ts_hbm.at[idx_s.at[i]], rows_s.at[b],
                                  sem_s[b]).wait()
            pltpu.sync_copy(rows_d.at[b], gd_hbm.at[pl.ds(base + i * _CH, _CH)])
            pltpu.sync_copy(rows_s.at[b], gs_hbm.at[pl.ds(base + i * _CH, _CH)])

        _pipelined(issue, drain_and_store, nchunk)

    return gather


@functools.cache
def _sc_scatter_kernel(n_edges):
    epw = n_edges // _NW
    nchunk = epw // _CH

    @functools.partial(
        pl.kernel,
        out_type=jax.ShapeDtypeStruct((_NC, _NPAD, NF), _F32),
        mesh=_sc_mesh(),
        scratch_types=[
            pltpu.VMEM((nchunk, _CH), jnp.int32),
            pltpu.VMEM((2, _CH, NF), _F32),
            pltpu.VMEM_SHARED((_NPAD, NF), _F32),
            pltpu.SemaphoreType.DMA,
            pltpu.SemaphoreType.DMA,
        ],
    )
    def scatter(eo_hbm, dst_hbm, zeros_hbm, out_hbm, idx_v, rows_v, acc,
                se0, se1):
        cid = lax.axis_index("c")
        sid = lax.axis_index("s")
        wid = sid * _NC + cid
        base = wid * epw
        pltpu.sync_copy(zeros_hbm.at[pl.ds(sid * _ROWS_PW, _ROWS_PW)],
                        acc.at[pl.ds(sid * _ROWS_PW, _ROWS_PW)])
        pltpu.sync_copy(dst_hbm.at[wid], idx_v)
        plsc.subcore_barrier()
        sem = (se0, se1)

        def load(i, b):
            pltpu.async_copy(eo_hbm.at[pl.ds(base + i * _CH, _CH)],
                             rows_v.at[b], sem[b])

        def drain_and_add(i, b):
            pltpu.make_async_copy(eo_hbm.at[pl.ds(base + i * _CH, _CH)],
                                  rows_v.at[b], sem[b]).wait()
            pltpu.sync_copy(rows_v.at[b], acc.at[idx_v.at[i]], add=True)

        _pipelined(load, drain_and_add, nchunk)
        plsc.subcore_barrier()
        pltpu.sync_copy(acc.at[pl.ds(sid * _ROWS_PW, _ROWS_PW)],
                        out_hbm.at[cid, pl.ds(sid * _ROWS_PW, _ROWS_PW)])

    return scatter


def _sc_gather(td, ts, dst3, src3, n_edges):
    return _sc_gather_kernel(n_edges)(td, ts, dst3, src3)


def _sc_scatter(eo, dst3, zeros_acc, n_edges):
    return _sc_scatter_kernel(n_edges)(eo, dst3, zeros_acc)


def kernel(x, edge_attr, lattice, params, edge_index, batch):
    p = params
    src, dst = edge_index[0], edge_index[1]
    blocks = []
    off = 0
    for ne in _EBLOCKS:
        d = lax.slice(dst, (off,), (off + ne,)).reshape(_NW, ne // (_NW * _CH), _CH)
        s = lax.slice(src, (off,), (off + ne,)).reshape(_NW, ne // (_NW * _CH), _CH)
        blocks.append((off, ne, d, s))
        off += ne

    lat_len = jnp.sqrt(jnp.sum(lattice * lattice, axis=-1))
    v1, v2, v3 = lattice[:, 0, :], lattice[:, 1, :], lattice[:, 2, :]
    n1, n2, n3 = lat_len[:, 0], lat_len[:, 1], lat_len[:, 2]
    cg = jnp.clip(jnp.sum(v1 * v2, axis=-1) / (n1 * n2), -1.0, 1.0)
    cb = jnp.clip(jnp.sum(v1 * v3, axis=-1) / (n1 * n3), -1.0, 1.0)
    ca = jnp.clip(jnp.sum(v2 * v3, axis=-1) / (n2 * n3), -1.0, 1.0)
    len_col = lat_len.reshape(3 * B, 1)
    ang_col = jnp.stack([cg, cb, ca], axis=1).reshape(3 * B, 1)

    e_len, e_ang = _tc_call(
        _lat_body,
        (jax.ShapeDtypeStruct((3 * B, NF), _F32),
         jax.ShapeDtypeStruct((3 * B, NF), _F32)),
    )(len_col, ang_col, p['lr_w1'], p['lr_b1'], p['lr_w2'], p['lr_b2'],
      p['la_w1'], p['la_b1'], p['la_w2'], p['la_b2'])

    lat_cat = jnp.concatenate(
        [e_len.reshape(B, 3 * NF), e_ang.reshape(B, 3 * NF)], axis=-1)
    lat_emb = _tc_call(
        _latemb_body, jax.ShapeDtypeStruct((B, NF), _F32),
    )(lat_cat, p['le_w1'], p['le_b1'], p['le_w2'], p['le_b2'])

    batch_col = batch.astype(_F32).reshape(N_NODES, 1)
    tile_n = 1000
    nf = _tc_call(
        _prelude_body, jax.ShapeDtypeStruct((N_NODES, NF), _F32),
        grid=(N_NODES // tile_n,),
        in_specs=[
            pl.BlockSpec((tile_n, x.shape[1]), lambda i: (i, 0)),
            pl.BlockSpec((tile_n, 1), lambda i: (i, 0)),
            pl.BlockSpec((B, NF), lambda i: (0, 0)),
            pl.BlockSpec(p['atom_w'].shape, lambda i: (0, 0)),
            pl.BlockSpec(p['atom_b'].shape, lambda i: (0,)),
            pl.BlockSpec(p['lae_w1'].shape, lambda i: (0, 0)),
            pl.BlockSpec(p['lae_b1'].shape, lambda i: (0,)),
            pl.BlockSpec(p['lae_w2'].shape, lambda i: (0, 0)),
            pl.BlockSpec(p['lae_b2'].shape, lambda i: (0,)),
        ],
        out_specs=pl.BlockSpec((tile_n, NF), lambda i: (i, 0)),
    )(x, batch_col, lat_emb, p['atom_w'], p['atom_b'],
      p['lae_w1'], p['lae_b1'], p['lae_w2'], p['lae_b2'])

    d2 = jnp.sum(edge_attr * edge_attr, axis=1).reshape(1, N_EDGES)
    c0, c1 = p['convs'][0], p['convs'][1]
    rb1c = p['rbf_b1'].reshape(-1, 1)
    rb2c = p['rbf_b2'].reshape(-1, 1)
    tile_e = 3200
    e1, e2 = _tc_call(
        _edgefeat_body,
        (jax.ShapeDtypeStruct((N_EDGES, NF), _BF16),
         jax.ShapeDtypeStruct((N_EDGES, NF), _BF16)),
        grid=(N_EDGES // tile_e,),
        in_specs=[
            pl.BlockSpec((1, tile_e), lambda i: (0, i)),
            pl.BlockSpec(p['rbf_w1'].shape, lambda i: (0, 0)),
            pl.BlockSpec(rb1c.shape, lambda i: (0, 0)),
            pl.BlockSpec(p['rbf_w2'].shape, lambda i: (0, 0)),
            pl.BlockSpec(rb2c.shape, lambda i: (0, 0)),
            pl.BlockSpec(c0['we'].shape, lambda i: (0, 0)),
            pl.BlockSpec(c0['be'].shape, lambda i: (0,)),
            pl.BlockSpec(c1['we'].shape, lambda i: (0, 0)),
            pl.BlockSpec(c1['be'].shape, lambda i: (0,)),
        ],
        out_specs=(pl.BlockSpec((tile_e, NF), lambda i: (i, 0)),
                   pl.BlockSpec((tile_e, NF), lambda i: (i, 0))),
    )(d2, p['rbf_w1'], rb1c, p['rbf_w2'], rb2c,
      c0['we'], c0['be'], c1['we'], c1['be'])

    zeros_acc = jnp.zeros((_NPAD, NF), _F32)

    for c, e_arr in ((c0, e1), (c1, e2)):
        td, ts = _tc_call(
            _qkv_body,
            (jax.ShapeDtypeStruct((N_NODES, 3 * NF), _F32),
             jax.ShapeDtypeStruct((N_NODES, 2 * NF), _F32)),
            grid=(N_NODES // tile_n,),
            in_specs=[
                pl.BlockSpec((tile_n, NF), lambda i: (i, 0)),
                pl.BlockSpec(c['wq'].shape, lambda i: (0, 0)),
                pl.BlockSpec(c['bq'].shape, lambda i: (0,)),
                pl.BlockSpec(c['wk'].shape, lambda i: (0, 0)),
                pl.BlockSpec(c['bk'].shape, lambda i: (0,)),
                pl.BlockSpec(c['wv'].shape, lambda i: (0, 0)),
                pl.BlockSpec(c['bv'].shape, lambda i: (0,)),
            ],
            out_specs=(pl.BlockSpec((tile_n, 3 * NF), lambda i: (i, 0)),
                       pl.BlockSpec((tile_n, 2 * NF), lambda i: (i, 0))),
        )(nf, c['wq'], c['bq'], c['wk'], c['bk'], c['wv'], c['bv'])

        gathered = [_sc_gather(td, ts, d, s, ne) for (off, ne, d, s) in blocks]

        tile_ec = 640
        aggs = []
        for (off, ne, d, s), (gd, gs) in zip(blocks, gathered):
            off_t = off // tile_ec
            eo = _tc_call(
                _edgecompute_body, jax.ShapeDtypeStruct((ne, NF), _F32),
                grid=(ne // tile_ec,),
                in_specs=[
                    pl.BlockSpec((tile_ec, 3 * NF), lambda i: (i, 0)),
                    pl.BlockSpec((tile_ec, 2 * NF), lambda i: (i, 0)),
                    pl.BlockSpec((tile_ec, NF), lambda i, o=off_t: (i + o, 0)),
                    pl.BlockSpec(c['wmu'].shape, lambda i: (0, 0)),
                    pl.BlockSpec(c['bmu'].shape, lambda i: (0,)),
                    pl.BlockSpec(c['wm'].shape, lambda i: (0, 0)),
                    pl.BlockSpec(c['bm'].shape, lambda i: (0,)),
                    pl.BlockSpec(c['ln_a_g'].shape, lambda i: (0,)),
                    pl.BlockSpec(c['ln_a_b'].shape, lambda i: (0,)),
                    pl.BlockSpec(c['ln_m_g'].shape, lambda i: (0,)),
                    pl.BlockSpec(c['ln_m_b'].shape, lambda i: (0,)),
                ],
                out_specs=pl.BlockSpec((tile_ec, NF), lambda i: (i, 0)),
            )(gd, gs, e_arr, c['wmu'].astype(_BF16), c['bmu'],
              c['wm'].astype(_BF16), c['bm'],
              c['ln_a_g'], c['ln_a_b'], c['ln_m_g'], c['ln_m_b'])
            aggs.append(_sc_scatter(eo, d, zeros_acc, ne))

        nf = _tc_call(
            _make_post_body(len(aggs)), jax.ShapeDtypeStruct((N_NODES, NF), _F32),
            grid=(1,),
            in_specs=(
                [pl.BlockSpec((_NC, N_NODES, NF), lambda i: (0, 0, 0))
                 for _ in aggs] +
                [pl.BlockSpec(c['wc'].shape, lambda i: (0, 0)),
                 pl.BlockSpec(c['bc'].shape, lambda i: (0,)),
                 pl.BlockSpec(c['bn_g'].shape, lambda i: (0,)),
                 pl.BlockSpec(c['bn_b'].shape, lambda i: (0,))]
            ),
            out_specs=pl.BlockSpec((N_NODES, NF), lambda i: (0, 0)),
        )(*aggs, c['wc'], c['bc'], c['bn_g'], c['bn_b'])

    batch_row = batch.astype(_F32).reshape(1, N_NODES)
    out = _tc_call(
        _readout_body, jax.ShapeDtypeStruct((B, PDOS), _F32),
    )(nf, batch_row, lat_emb, p['fc_w'], p['fc_b'], p['pdos_w'], p['pdos_b'])
    return out


# speedup vs baseline: 1.2072x; 1.2072x over previous
import functools

import jax
import jax.numpy as jnp
import numpy as np
from jax import lax
from jax.experimental import pallas as pl
from jax.experimental.pallas import tpu as pltpu
from jax.experimental.pallas import tpu_sc as plsc

N_NODES = 10000
N_EDGES = 160000
B = 64
NF = 128
EDGE_BINS = 128
TRIP = 40
PDOS = 200

_NC = 2
_NS = 16
_NW = _NC * _NS
_CH = 40
_NPAD = 10240
_ROWS_PW = _NPAD // _NS

_EBLOCKS = (81920, 78080)

_F32 = jnp.float32
_BF16 = jnp.bfloat16


def _softplus(x):
    return jnp.log1p(jnp.exp(-jnp.abs(x))) + jnp.maximum(x, 0.0)


def _silu(x):
    return x * jax.nn.sigmoid(x)


def _lnorm(x, g, b, eps=1e-5):
    m = jnp.mean(x, axis=-1, keepdims=True)
    v = jnp.mean((x - m) ** 2, axis=-1, keepdims=True)
    return (x - m) * lax.rsqrt(v + eps) * g + b


def _rbf_rows(col, vmin, vmax, bins):
    step = (vmax - vmin) / (bins - 1)
    centers = vmin + step * lax.broadcasted_iota(jnp.int32, (1, bins), 1).astype(_F32)
    gamma = 1.0 / (step * step)
    return jnp.exp(-gamma * (col - centers) ** 2)


def _lat_body(len_ref, ang_ref, rw1, rb1, rw2, rb2, aw1, ab1, aw2, ab2,
              elen_ref, eang_ref):
    r = _rbf_rows(len_ref[...], 0.0, 8.0, EDGE_BINS)
    elen_ref[...] = _softplus(r @ rw1[...] + rb1[...][None]) @ rw2[...] + rb2[...][None]
    a = _rbf_rows(ang_ref[...], -1.0, 1.0, TRIP)
    eang_ref[...] = _softplus(a @ aw1[...] + ab1[...][None]) @ aw2[...] + ab2[...][None]


def _latemb_body(cat_ref, w1, b1, w2, b2, out_ref):
    h = _softplus(cat_ref[...] @ w1[...] + b1[...][None]) @ w2[...] + b2[...][None]
    out_ref[...] = h


def _prelude_body(x_ref, b_ref, lat_ref, aw, ab, w1, b1, w2, b2, out_ref):
    nf0 = x_ref[...] @ aw[...] + ab[...][None]
    rows = x_ref.shape[0]
    onehot = (b_ref[...] == lax.broadcasted_iota(jnp.int32, (rows, B), 1).astype(_F32)).astype(_F32)
    le = onehot @ lat_ref[...]
    h = jnp.concatenate([nf0, le], axis=-1)
    out_ref[...] = _softplus(h @ w1[...] + b1[...][None]) @ w2[...] + b2[...][None]


def _edgefeat_body(d2_ref, rw1, rb1c, rw2, rb2c, we1, be1, we2, be2,
                   e1_ref, e2_ref):
    d = jnp.sqrt(d2_ref[...])
    step = 8.0 / (EDGE_BINS - 1)
    centers = step * lax.broadcasted_iota(jnp.int32, (EDGE_BINS, 1), 0).astype(_F32)
    gamma = 1.0 / (step * step)
    rt = jnp.exp(-gamma * (d - centers) ** 2)
    ht = _softplus(jax.lax.dot_general(
        rw1[...], rt, (((0,), (0,)), ((), ())),
        preferred_element_type=_F32) + rb1c[...])
    eft = jax.lax.dot_general(
        rw2[...], ht, (((0,), (0,)), ((), ())),
        preferred_element_type=_F32) + rb2c[...]
    e1 = jax.lax.dot_general(eft, we1[...], (((0,), (0,)), ((), ())),
                             preferred_element_type=_F32)
    e2 = jax.lax.dot_general(eft, we2[...], (((0,), (0,)), ((), ())),
                             preferred_element_type=_F32)
    e1_ref[...] = (e1 + be1[...][None]).astype(jnp.bfloat16)
    e2_ref[...] = (e2 + be2[...][None]).astype(jnp.bfloat16)


def _pack2(hi, lo):
    hb = pltpu.bitcast(hi.astype(_BF16), jnp.uint16).astype(jnp.uint32)
    lb = pltpu.bitcast(lo.astype(_BF16), jnp.uint16).astype(jnp.uint32)
    return pltpu.bitcast((hb << 16) | lb, jnp.int32)


def _unpack2(w):
    u = pltpu.bitcast(w, jnp.uint32)
    hi = pltpu.bitcast((u >> 16).astype(jnp.uint16), _BF16)
    lo = pltpu.bitcast((u & 0xFFFF).astype(jnp.uint16), _BF16)
    return hi, lo


def _qkv_body(nf_ref, wq, bq, wk, bk, wv, bv, td_ref, ts_ref):
    nf = nf_ref[...]
    q = nf @ wq[...] + bq[...][None]
    k = nf @ wk[...] + bk[...][None]
    v = nf @ wv[...] + bv[...][None]
    td_ref[...] = jnp.concatenate([_pack2(q, q * k), _pack2(v, v)], axis=-1)
    ts_ref[...] = _pack2(k, v)


def _edgecompute_body(gd_ref, gs_ref, e_ref, wmu, bmu, wm, bm,
                      lag, lab, lmg, lmb, out_ref):
    gd = gd_ref[...]
    e_b = e_ref[...]
    q_i, qk_i = _unpack2(gd[:, :NF])
    v_i, _ = _unpack2(gd[:, NF:])
    k_j, v_j = _unpack2(gs_ref[...])
    e = e_b.astype(_F32)
    q_f = q_i.astype(_F32)
    alpha = jnp.concatenate([qk_i.astype(_F32),
                             q_f * k_j.astype(_F32), q_f * e], axis=-1)
    alpha = alpha * np.float32(1.0 / np.sqrt(3.0 * NF))
    gate = jax.nn.sigmoid(_lnorm(alpha, lag[...][None], lab[...][None]))
    msg = jnp.dot(jnp.concatenate([v_i, v_j, e_b], axis=-1), wmu[...],
                  preferred_element_type=_F32) + bmu[...][None]
    msg = msg * gate
    h = jnp.dot(msg.astype(jnp.bfloat16), wm[...],
                preferred_element_type=_F32) + bm[...][None]
    out_ref[...] = _lnorm(h, lmg[...][None], lmb[...][None])


def _make_post_body(n_blocks):
    def body(*args):
        agg_refs = args[:n_blocks]
        wc, bc, bng, bnb, out_ref = args[n_blocks:]
        agg = agg_refs[0][0] + agg_refs[0][1]
        for a in agg_refs[1:]:
            agg = agg + a[0] + a[1]
        o = agg @ wc[...] + bc[...][None]
        m = jnp.mean(o, axis=0, keepdims=True)
        v = jnp.mean((o - m) ** 2, axis=0, keepdims=True)
        o = (o - m) * lax.rsqrt(v + 1e-5) * bng[...][None] + bnb[...][None]
        out_ref[...] = _silu(o)
    return body


def _readout_body(nf_ref, brow_ref, lat_ref, fcw, fcb, pw, pb, out_ref):
    nf = nf_ref[...]
    onehot_t = (brow_ref[...] ==
                lax.broadcasted_iota(jnp.int32, (B, N_NODES), 0).astype(_F32)
                ).astype(_F32)
    seg = onehot_t @ nf
    cnt = jnp.sum(onehot_t, axis=1, keepdims=True)
    feats = seg / jnp.maximum(cnt, 1.0) + lat_ref[...]
    h = _silu(feats @ fcw[...] + fcb[...][None])
    out_ref[...] = h @ pw[...] + pb[...][None]


def _tc_call(body, out_shapes, grid=None, in_specs=None, out_specs=None):
    kwargs = {}
    if grid is not None:
        kwargs["grid"] = grid
    if in_specs is not None:
        kwargs["in_specs"] = in_specs
    if out_specs is not None:
        kwargs["out_specs"] = out_specs
    return pl.pallas_call(body, out_shape=out_shapes, **kwargs)


@functools.cache
def _sc_mesh():
    return plsc.VectorSubcoreMesh(core_axis_name="c", subcore_axis_name="s",
                                  num_cores=_NC, num_subcores=_NS)


def _pipelined(issue, drain, nchunk):
    issue(0, 0)
    npair = (nchunk - 1) // 2 if nchunk % 2 else (nchunk - 2) // 2

    @pl.loop(0, npair)
    def pair(j):
        i0 = 2 * j
        issue(i0 + 1, 1)
        drain(i0, 0)
        issue(i0 + 2, 0)
        drain(i0 + 1, 1)

    if nchunk % 2:
        drain(nchunk - 1, 0)
    else:
        issue(nchunk - 1, 1)
        drain(nchunk - 2, 0)
        drain(nchunk - 1, 1)


@functools.cache
def _sc_gather_kernel(n_edges):
    epw = n_edges // _NW
    nchunk = epw // _CH

    @functools.partial(
        pl.kernel,
        out_type=(
            jax.ShapeDtypeStruct((n_edges, 2 * NF), jnp.int32),
            jax.ShapeDtypeStruct((n_edges, NF), jnp.int32),
        ),
        mesh=_sc_mesh(),
        scratch_types=[
            pltpu.VMEM((nchunk, _CH), jnp.int32),
            pltpu.VMEM((nchunk, _CH), jnp.int32),
            pltpu.VMEM((2, _CH, 2 * NF), jnp.int32),
            pltpu.VMEM((2, _CH, NF), jnp.int32),
            pltpu.SemaphoreType.DMA,
            pltpu.SemaphoreType.DMA,
            pltpu.SemaphoreType.DMA,
            pltpu.SemaphoreType.DMA,
        ],
    )
    def gather(td_hbm, ts_hbm, dst_hbm, src_hbm, gd_hbm, gs_hbm,
               idx_d, idx_s, rows_d, rows_s, sd0, sd1, ss0, ss1):
        wid = lax.axis_index("s") * _NC + lax.axis_index("c")
        base = wid * epw
        pltpu.sync_copy(dst_hbm.at[wid], idx_d)
        pltpu.sync_copy(src_hbm.at[wid], idx_s)
        sem_d = (sd0, sd1)
        sem_s = (ss0, ss1)

        def issue(i, b):
            pltpu.async_copy(td_hbm.at[idx_d.at[i]], rows_d.at[b], sem_d[b])
            pltpu.async_copy(ts_hbm.at[idx_s.at[i]], rows_s.at[b], sem_s[b])

        def drain_and_store(i, b):
            pltpu.make_async_copy(td_hbm.at[idx_d.at[i]], rows_d.at[b],
                                  sem_d[b]).wait()
            pltpu.make_async_copy(ts_hbm.at[idx_s.at[i]], rows_s.at[b],
                                  sem_s[b]).wait()
            pltpu.sync_copy(rows_d.at[b], gd_hbm.at[pl.ds(base + i * _CH, _CH)])
            pltpu.sync_copy(rows_s.at[b], gs_hbm.at[pl.ds(base + i * _CH, _CH)])

        _pipelined(issue, drain_and_store, nchunk)

    return gather


@functools.cache
def _sc_scatter_kernel(n_edges):
    epw = n_edges // _NW
    nchunk = epw // _CH

    @functools.partial(
        pl.kernel,
        out_type=jax.ShapeDtypeStruct((_NC, _NPAD, NF), _F32),
        mesh=_sc_mesh(),
        scratch_types=[
            pltpu.VMEM((nchunk, _CH), jnp.int32),
            pltpu.VMEM((2, _CH, NF), _F32),
            pltpu.VMEM_SHARED((_NPAD, NF), _F32),
            pltpu.SemaphoreType.DMA,
            pltpu.SemaphoreType.DMA,
        ],
    )
    def scatter(eo_hbm, dst_hbm, zeros_hbm, out_hbm, idx_v, rows_v, acc,
                se0, se1):
        cid = lax.axis_index("c")
        sid = lax.axis_index("s")
        wid = sid * _NC + cid
        base = wid * epw
        pltpu.sync_copy(zeros_hbm.at[pl.ds(sid * _ROWS_PW, _ROWS_PW)],
                        acc.at[pl.ds(sid * _ROWS_PW, _ROWS_PW)])
        pltpu.sync_copy(dst_hbm.at[wid], idx_v)
        plsc.subcore_barrier()
        sem = (se0, se1)

        def load(i, b):
            pltpu.async_copy(eo_hbm.at[pl.ds(base + i * _CH, _CH)],
                             rows_v.at[b], sem[b])

        def drain_and_add(i, b):
            pltpu.make_async_copy(eo_hbm.at[pl.ds(base + i * _CH, _CH)],
                                  rows_v.at[b], sem[b]).wait()
            pltpu.sync_copy(rows_v.at[b], acc.at[idx_v.at[i]], add=True)

        _pipelined(load, drain_and_add, nchunk)
        plsc.subcore_barrier()
        pltpu.sync_copy(acc.at[pl.ds(sid * _ROWS_PW, _ROWS_PW)],
                        out_hbm.at[cid, pl.ds(sid * _ROWS_PW, _ROWS_PW)])

    return scatter


def _sc_gather(td, ts, dst3, src3, n_edges):
    return _sc_gather_kernel(n_edges)(td, ts, dst3, src3)


def _sc_scatter(eo, dst3, zeros_acc, n_edges):
    return _sc_scatter_kernel(n_edges)(eo, dst3, zeros_acc)


def kernel(x, edge_attr, lattice, params, edge_index, batch):
    p = params
    src, dst = edge_index[0], edge_index[1]
    blocks = []
    off = 0
    for ne in _EBLOCKS:
        d = lax.slice(dst, (off,), (off + ne,)).reshape(_NW, ne // (_NW * _CH), _CH)
        s = lax.slice(src, (off,), (off + ne,)).reshape(_NW, ne // (_NW * _CH), _CH)
        blocks.append((off, ne, d, s))
        off += ne

    lat_len = jnp.sqrt(jnp.sum(lattice * lattice, axis=-1))
    v1, v2, v3 = lattice[:, 0, :], lattice[:, 1, :], lattice[:, 2, :]
    n1, n2, n3 = lat_len[:, 0], lat_len[:, 1], lat_len[:, 2]
    cg = jnp.clip(jnp.sum(v1 * v2, axis=-1) / (n1 * n2), -1.0, 1.0)
    cb = jnp.clip(jnp.sum(v1 * v3, axis=-1) / (n1 * n3), -1.0, 1.0)
    ca = jnp.clip(jnp.sum(v2 * v3, axis=-1) / (n2 * n3), -1.0, 1.0)
    len_col = lat_len.reshape(3 * B, 1)
    ang_col = jnp.stack([cg, cb, ca], axis=1).reshape(3 * B, 1)

    e_len, e_ang = _tc_call(
        _lat_body,
        (jax.ShapeDtypeStruct((3 * B, NF), _F32),
         jax.ShapeDtypeStruct((3 * B, NF), _F32)),
    )(len_col, ang_col, p['lr_w1'], p['lr_b1'], p['lr_w2'], p['lr_b2'],
      p['la_w1'], p['la_b1'], p['la_w2'], p['la_b2'])

    lat_cat = jnp.concatenate(
        [e_len.reshape(B, 3 * NF), e_ang.reshape(B, 3 * NF)], axis=-1)
    lat_emb = _tc_call(
        _latemb_body, jax.ShapeDtypeStruct((B, NF), _F32),
    )(lat_cat, p['le_w1'], p['le_b1'], p['le_w2'], p['le_b2'])

    batch_col = batch.astype(_F32).reshape(N_NODES, 1)
    tile_n = 1000
    nf = _tc_call(
        _prelude_body, jax.ShapeDtypeStruct((N_NODES, NF), _F32),
        grid=(N_NODES // tile_n,),
        in_specs=[
            pl.BlockSpec((tile_n, x.shape[1]), lambda i: (i, 0)),
            pl.BlockSpec((tile_n, 1), lambda i: (i, 0)),
            pl.BlockSpec((B, NF), lambda i: (0, 0)),
            pl.BlockSpec(p['atom_w'].shape, lambda i: (0, 0)),
            pl.BlockSpec(p['atom_b'].shape, lambda i: (0,)),
            pl.BlockSpec(p['lae_w1'].shape, lambda i: (0, 0)),
            pl.BlockSpec(p['lae_b1'].shape, lambda i: (0,)),
            pl.BlockSpec(p['lae_w2'].shape, lambda i: (0, 0)),
            pl.BlockSpec(p['lae_b2'].shape, lambda i: (0,)),
        ],
        out_specs=pl.BlockSpec((tile_n, NF), lambda i: (i, 0)),
    )(x, batch_col, lat_emb, p['atom_w'], p['atom_b'],
      p['lae_w1'], p['lae_b1'], p['lae_w2'], p['lae_b2'])

    d2 = jnp.sum(edge_attr * edge_attr, axis=1).reshape(1, N_EDGES)
    c0, c1 = p['convs'][0], p['convs'][1]
    rb1c = p['rbf_b1'].reshape(-1, 1)
    rb2c = p['rbf_b2'].reshape(-1, 1)
    tile_e = 3200
    e1, e2 = _tc_call(
        _edgefeat_body,
        (jax.ShapeDtypeStruct((N_EDGES, NF), _BF16),
         jax.ShapeDtypeStruct((N_EDGES, NF), _BF16)),
        grid=(N_EDGES // tile_e,),
        in_specs=[
            pl.BlockSpec((1, tile_e), lambda i: (0, i)),
            pl.BlockSpec(p['rbf_w1'].shape, lambda i: (0, 0)),
            pl.BlockSpec(rb1c.shape, lambda i: (0, 0)),
            pl.BlockSpec(p['rbf_w2'].shape, lambda i: (0, 0)),
            pl.BlockSpec(rb2c.shape, lambda i: (0, 0)),
            pl.BlockSpec(c0['we'].shape, lambda i: (0, 0)),
            pl.BlockSpec(c0['be'].shape, lambda i: (0,)),
            pl.BlockSpec(c1['we'].shape, lambda i: (0, 0)),
            pl.BlockSpec(c1['be'].shape, lambda i: (0,)),
        ],
        out_specs=(pl.BlockSpec((tile_e, NF), lambda i: (i, 0)),
                   pl.BlockSpec((tile_e, NF), lambda i: (i, 0))),
    )(d2, p['rbf_w1'], rb1c, p['rbf_w2'], rb2c,
      c0['we'], c0['be'], c1['we'], c1['be'])

    zeros_acc = jnp.zeros((_NPAD, NF), _F32)

    for c, e_arr in ((c0, e1), (c1, e2)):
        td, ts = _tc_call(
            _qkv_body,
            (jax.ShapeDtypeStruct((N_NODES, 2 * NF), jnp.int32),
             jax.ShapeDtypeStruct((N_NODES, NF), jnp.int32)),
            grid=(N_NODES // tile_n,),
            in_specs=[
                pl.BlockSpec((tile_n, NF), lambda i: (i, 0)),
                pl.BlockSpec(c['wq'].shape, lambda i: (0, 0)),
                pl.BlockSpec(c['bq'].shape, lambda i: (0,)),
                pl.BlockSpec(c['wk'].shape, lambda i: (0, 0)),
                pl.BlockSpec(c['bk'].shape, lambda i: (0,)),
                pl.BlockSpec(c['wv'].shape, lambda i: (0, 0)),
                pl.BlockSpec(c['bv'].shape, lambda i: (0,)),
            ],
            out_specs=(pl.BlockSpec((tile_n, 2 * NF), lambda i: (i, 0)),
                       pl.BlockSpec((tile_n, NF), lambda i: (i, 0))),
        )(nf, c['wq'], c['bq'], c['wk'], c['bk'], c['wv'], c['bv'])

        gathered = [_sc_gather(td, ts, d, s, ne) for (off, ne, d, s) in blocks]

        tile_ec = 640
        aggs = []
        for (off, ne, d, s), (gd, gs) in zip(blocks, gathered):
            off_t = off // tile_ec
            eo = _tc_call(
                _edgecompute_body, jax.ShapeDtypeStruct((ne, NF), _F32),
                grid=(ne // tile_ec,),
                in_specs=[
                    pl.BlockSpec((tile_ec, 2 * NF), lambda i: (i, 0)),
                    pl.BlockSpec((tile_ec, NF), lambda i: (i, 0)),
                    pl.BlockSpec((tile_ec, NF), lambda i, o=off_t: (i + o, 0)),
                    pl.BlockSpec(c['wmu'].shape, lambda i: (0, 0)),
                    pl.BlockSpec(c['bmu'].shape, lambda i: (0,)),
                    pl.BlockSpec(c['wm'].shape, lambda i: (0, 0)),
                    pl.BlockSpec(c['bm'].shape, lambda i: (0,)),
                    pl.BlockSpec(c['ln_a_g'].shape, lambda i: (0,)),
                    pl.BlockSpec(c['ln_a_b'].shape, lambda i: (0,)),
                    pl.BlockSpec(c['ln_m_g'].shape, lambda i: (0,)),
                    pl.BlockSpec(c['ln_m_b'].shape, lambda i: (0,)),
                ],
                out_specs=pl.BlockSpec((tile_ec, NF), lambda i: (i, 0)),
            )(gd, gs, e_arr, c['wmu'].astype(_BF16), c['bmu'],
              c['wm'].astype(_BF16), c['bm'],
              c['ln_a_g'], c['ln_a_b'], c['ln_m_g'], c['ln_m_b'])
            aggs.append(_sc_scatter(eo, d, zeros_acc, ne))

        nf = _tc_call(
            _make_post_body(len(aggs)), jax.ShapeDtypeStruct((N_NODES, NF), _F32),
            grid=(1,),
            in_specs=(
                [pl.BlockSpec((_NC, N_NODES, NF), lambda i: (0, 0, 0))
                 for _ in aggs] +
                [pl.BlockSpec(c['wc'].shape, lambda i: (0, 0)),
                 pl.BlockSpec(c['bc'].shape, lambda i: (0,)),
                 pl.BlockSpec(c['bn_g'].shape, lambda i: (0,)),
                 pl.BlockSpec(c['bn_b'].shape, lambda i: (0,))]
            ),
            out_specs=pl.BlockSpec((N_NODES, NF), lambda i: (0, 0)),
        )(*aggs, c['wc'], c['bc'], c['bn_g'], c['bn_b'])

    batch_row = batch.astype(_F32).reshape(1, N_NODES)
    out = _tc_call(
        _readout_body, jax.ShapeDtypeStruct((B, PDOS), _F32),
    )(nf, batch_row, lat_emb, p['fc_w'], p['fc_b'], p['pdos_w'], p['pdos_b'])
    return out
